# Initial kernel scaffold; baseline (speedup 1.0000x reference)
import jax
import jax.numpy as jnp
from jax.experimental import pallas as pl


def kernel(input_ids, bbox, token_type_ids, word_emb, x_pos_emb, y_pos_emb, h_pos_emb, w_pos_emb, token_type_emb, pos_emb, ln_gamma, ln_beta):
    raise NotImplementedError("write your pallas kernel here")


# speedup vs baseline: 1.3181x; 1.3181x over previous
import functools

import jax
import jax.numpy as jnp
from jax import lax
from jax.experimental import pallas as pl
from jax.experimental.pallas import tpu as pltpu
from jax.experimental.pallas import tpu_sc as plsc

H = 768
NJ = H // 16
NW = 32
NTOK = 64 * 512
TPW = NTOK // NW
GK = 8
NG = TPW // GK
F32 = jnp.float32
I32 = jnp.int32


def _mesh():
    return plsc.VectorSubcoreMesh(core_axis_name="c", subcore_axis_name="s")


@functools.partial(
    pl.kernel,
    out_type=jax.ShapeDtypeStruct((NTOK, H), F32),
    mesh=_mesh(),
    scratch_types=[
        pltpu.VMEM((GK, H), F32), pltpu.VMEM((GK, H), F32),
        pltpu.VMEM((2 * GK, H), F32), pltpu.VMEM((2 * GK, H), F32),
        pltpu.VMEM((2 * GK, H), F32), pltpu.VMEM((2 * GK, H), F32),
        pltpu.VMEM((2 * GK, H), F32), pltpu.VMEM((2 * GK, H), F32),
        pltpu.VMEM((GK, H), F32), pltpu.VMEM((GK, H), F32),
        pltpu.VMEM((GK, H), F32), pltpu.VMEM((GK, H), F32),
        pltpu.VMEM((TPW,), I32),
        pltpu.VMEM((4 * TPW,), I32),
        pltpu.VMEM((TPW,), I32),
        pltpu.VMEM((16,), I32), pltpu.VMEM((16,), I32),
        pltpu.VMEM((16,), I32), pltpu.VMEM((16,), I32),
        pltpu.VMEM((16,), I32), pltpu.VMEM((16,), I32),
        pltpu.VMEM((2, H), F32),
        pltpu.VMEM((H,), F32),
        pltpu.VMEM((H,), F32),
        pltpu.VMEM((H,), F32),
        pltpu.SemaphoreType.DMA, pltpu.SemaphoreType.DMA,
        pltpu.SemaphoreType.DMA, pltpu.SemaphoreType.DMA,
    ],
    compiler_params=pltpu.CompilerParams(needs_layout_passes=False),
)
def _emb_ln(word_hbm, x_hbm, y_hbm, hw_hbm, tt_hbm, pos_hbm,
            ids_hbm, bb_hbm, ttid_hbm, gamma_hbm, beta_hbm, out_hbm,
            rw0, rw1, rx0, rx1, ry0, ry1, rhw0, rhw1, pb0, pb1, ac0, ac1,
            idsv, bbv, ttv, xi0, xi1, yi0, yi1, hi0, hi1,
            ttr, dlt, gv, bv, sg0, sg1, so0, so1):
    wid = lax.axis_index("s") * 2 + lax.axis_index("c")
    base = pl.multiple_of(wid * TPW, TPW)

    pltpu.sync_copy(ids_hbm.at[pl.ds(base, TPW)], idsv)
    pltpu.sync_copy(bb_hbm.at[pl.ds(pl.multiple_of(base * 4, 4 * TPW), 4 * TPW)], bbv)
    pltpu.sync_copy(ttid_hbm.at[pl.ds(base, TPW)], ttv)
    pltpu.sync_copy(tt_hbm, ttr)
    pltpu.sync_copy(gamma_hbm, gv)
    pltpu.sync_copy(beta_hbm, bv)

    def _dlt(j, c):
        sl = pl.ds(j * 16, 16)
        dlt[sl] = ttr[1, sl] - ttr[0, sl]
        return c
    lax.fori_loop(0, NJ, _dlt, 0)

    io = lax.iota(I32, 16)
    i8 = jnp.bitwise_and(io, 7)
    lo = io < 8

    def fire(g, rw, rx, ry, rhw, pb, xi, yi, hi, sg):
        g8 = pl.multiple_of(g * GK, GK)
        tb = (g8 + i8) * 4
        xi[...] = plsc.load_gather(bbv, [tb + jnp.where(lo, 0, 2)])
        yi[...] = plsc.load_gather(bbv, [tb + jnp.where(lo, 1, 3)])
        bhi = plsc.load_gather(bbv, [tb + jnp.where(lo, 3, 2)])
        blo = plsc.load_gather(bbv, [tb + jnp.where(lo, 1, 0)])
        hi[...] = bhi - blo + jnp.where(lo, 0, 1024)
        pltpu.async_copy(word_hbm.at[idsv.at[pl.ds(g8, GK)]], rw, sg)
        pltpu.async_copy(x_hbm.at[xi], rx, sg)
        pltpu.async_copy(y_hbm.at[yi], ry, sg)
        pltpu.async_copy(hw_hbm.at[hi], rhw, sg)
        s0 = pl.multiple_of(jnp.bitwise_and(g8, 511), GK)
        pltpu.async_copy(pos_hbm.at[pl.ds(s0, GK)], pb, sg)

    def wait_gathers(rw, rx, ry, rhw, pb, sg):
        pltpu.make_async_copy(word_hbm.at[pl.ds(0, GK)], rw, sg).wait()
        pltpu.make_async_copy(x_hbm.at[pl.ds(0, 2 * GK)], rx, sg).wait()
        pltpu.make_async_copy(y_hbm.at[pl.ds(0, 2 * GK)], ry, sg).wait()
        pltpu.make_async_copy(hw_hbm.at[pl.ds(0, 2 * GK)], rhw, sg).wait()
        pltpu.make_async_copy(pos_hbm.at[pl.ds(0, GK)], pb, sg).wait()

    def wait_out(ac, so):
        pltpu.make_async_copy(ac, out_hbm.at[pl.ds(base, GK)], so).wait()

    def compute(g, rw, rx, ry, rhw, pb, ac):
        g8 = g * GK

        def tbody(t, ct):
            ttf = plsc.load_gather(ttv, [jnp.full((16,), g8 + t, I32)]
                                   ).astype(F32)

            def sbody(j, carry):
                sv, qv = carry
                sl = pl.ds(j * 16, 16)
                s = pb[t, sl] + ttr[0, sl] + ttf * dlt[sl]
                s = s + rw[t, sl]
                s = s + rx[t, sl] + rx[t + GK, sl]
                s = s + ry[t, sl] + ry[t + GK, sl]
                s = s + rhw[t, sl] + rhw[t + GK, sl]
                ac[t, sl] = s
                return sv + s, qv + s * s

            z = jnp.zeros((16,), F32)
            sv, qv = lax.fori_loop(0, NJ, sbody, (z, z))
            mv = lax.broadcast_in_dim(jnp.sum(sv), (16,), ()) * (1.0 / H)
            q2 = lax.broadcast_in_dim(jnp.sum(qv), (16,), ()) * (1.0 / H)
            var = q2 - mv * mv + 1e-5
            iv = plsc.bitcast(var, I32)
            yv = plsc.bitcast(jnp.int32(0x5F3759DF) - (iv >> 1), F32)
            for _ in range(3):
                yv = yv * (1.5 - 0.5 * var * yv * yv)
            bc = -mv * yv

            def nbody(j, c):
                sl = pl.ds(j * 16, 16)
                ac[t, sl] = (ac[t, sl] * yv + bc) * gv[sl] + bv[sl]
                return c
            lax.fori_loop(0, NJ, nbody, 0)
            return ct
        lax.fori_loop(0, GK, tbody, 0)

    def fire_out(g, ac, so):
        off = pl.multiple_of(base + g * GK, GK)
        pltpu.async_copy(ac, out_hbm.at[pl.ds(off, GK)], so)

    fire(0, rw0, rx0, ry0, rhw0, pb0, xi0, yi0, hi0, sg0)

    def ibody(i, c):
        g0 = 2 * i
        g1 = g0 + 1
        fire(g1, rw1, rx1, ry1, rhw1, pb1, xi1, yi1, hi1, sg1)
        wait_gathers(rw0, rx0, ry0, rhw0, pb0, sg0)

        @pl.when(i > 0)
        def _():
            wait_out(ac0, so0)
        compute(g0, rw0, rx0, ry0, rhw0, pb0, ac0)
        fire_out(g0, ac0, so0)

        @pl.when(i < NG // 2 - 1)
        def _():
            fire(g0 + 2, rw0, rx0, ry0, rhw0, pb0, xi0, yi0, hi0, sg0)
        wait_gathers(rw1, rx1, ry1, rhw1, pb1, sg1)

        @pl.when(i > 0)
        def _():
            wait_out(ac1, so1)
        compute(g1, rw1, rx1, ry1, rhw1, pb1, ac1)
        fire_out(g1, ac1, so1)
        return c
    lax.fori_loop(0, NG // 2, ibody, 0)
    wait_out(ac0, so0)
    wait_out(ac1, so1)


def kernel(input_ids, bbox, token_type_ids, word_emb, x_pos_emb, y_pos_emb,
           h_pos_emb, w_pos_emb, token_type_emb, pos_emb, ln_gamma, ln_beta):
    b, s = input_ids.shape
    ids = input_ids.reshape(-1).astype(I32)
    bb = bbox.reshape(-1).astype(I32)
    tt = token_type_ids.reshape(-1).astype(I32)
    hw = jnp.concatenate([h_pos_emb, w_pos_emb], axis=0)
    out = _emb_ln(word_emb, x_pos_emb, y_pos_emb, hw, token_type_emb,
                  pos_emb[:512], ids, bb, tt, ln_gamma, ln_beta)
    return out.reshape(b, s, H)


# speedup vs baseline: 3.4515x; 2.6186x over previous
import functools

import jax
import jax.numpy as jnp
from jax import lax
from jax.experimental import pallas as pl
from jax.experimental.pallas import tpu as pltpu
from jax.experimental.pallas import tpu_sc as plsc

H = 768
NJ = H // 16
NW = 32
NTOK = 64 * 512
TPW = NTOK // NW
GK = 8
NG = TPW // GK
F32 = jnp.float32
I32 = jnp.int32


def _mesh():
    return plsc.VectorSubcoreMesh(core_axis_name="c", subcore_axis_name="s")


@functools.partial(
    pl.kernel,
    out_type=jax.ShapeDtypeStruct((NTOK, H), F32),
    mesh=_mesh(),
    scratch_types=[
        pltpu.VMEM((GK, H), F32), pltpu.VMEM((GK, H), F32),
        pltpu.VMEM((2 * GK, H), F32), pltpu.VMEM((2 * GK, H), F32),
        pltpu.VMEM((2 * GK, H), F32), pltpu.VMEM((2 * GK, H), F32),
        pltpu.VMEM((2 * GK, H), F32), pltpu.VMEM((2 * GK, H), F32),
        pltpu.VMEM((GK, H), F32), pltpu.VMEM((GK, H), F32),
        pltpu.VMEM((GK, H), F32), pltpu.VMEM((GK, H), F32),
        pltpu.VMEM((TPW,), I32),
        pltpu.VMEM((4 * TPW,), I32),
        pltpu.VMEM((TPW,), I32),
        pltpu.VMEM((16,), I32), pltpu.VMEM((16,), I32),
        pltpu.VMEM((16,), I32), pltpu.VMEM((16,), I32),
        pltpu.VMEM((16,), I32), pltpu.VMEM((16,), I32),
        pltpu.VMEM((2, H), F32),
        pltpu.VMEM((H,), F32),
        pltpu.VMEM((H,), F32),
        pltpu.VMEM((H,), F32),
        pltpu.SemaphoreType.DMA, pltpu.SemaphoreType.DMA,
        pltpu.SemaphoreType.DMA, pltpu.SemaphoreType.DMA,
    ],
    compiler_params=pltpu.CompilerParams(needs_layout_passes=False),
)
def _emb_ln(word_hbm, x_hbm, y_hbm, hw_hbm, tt_hbm, pos_hbm,
            ids_hbm, bb_hbm, ttid_hbm, gamma_hbm, beta_hbm, out_hbm,
            rw0, rw1, rx0, rx1, ry0, ry1, rhw0, rhw1, pb0, pb1, ac0, ac1,
            idsv, bbv, ttv, xi0, xi1, yi0, yi1, hi0, hi1,
            ttr, dlt, gv, bv, sg0, sg1, so0, so1):
    wid = lax.axis_index("s") * 2 + lax.axis_index("c")
    base = pl.multiple_of(wid * TPW, TPW)

    pltpu.sync_copy(ids_hbm.at[pl.ds(base, TPW)], idsv)
    pltpu.sync_copy(bb_hbm.at[pl.ds(pl.multiple_of(base * 4, 4 * TPW), 4 * TPW)], bbv)
    pltpu.sync_copy(ttid_hbm.at[pl.ds(base, TPW)], ttv)
    pltpu.sync_copy(tt_hbm, ttr)
    pltpu.sync_copy(gamma_hbm, gv)
    pltpu.sync_copy(beta_hbm, bv)

    def _dlt(j, c):
        sl = pl.ds(j * 16, 16)
        dlt[sl] = ttr[1, sl] - ttr[0, sl]
        return c
    lax.fori_loop(0, NJ, _dlt, 0)

    io = lax.iota(I32, 16)
    i8 = jnp.bitwise_and(io, 7)
    lo = io < 8

    def fire(g, rw, rx, ry, rhw, pb, xi, yi, hi, sg):
        g8 = pl.multiple_of(g * GK, GK)
        tb = (g8 + i8) * 4
        xi[...] = plsc.load_gather(bbv, [tb + jnp.where(lo, 0, 2)])
        yi[...] = plsc.load_gather(bbv, [tb + jnp.where(lo, 1, 3)])
        bhi = plsc.load_gather(bbv, [tb + jnp.where(lo, 3, 2)])
        blo = plsc.load_gather(bbv, [tb + jnp.where(lo, 1, 0)])
        hi[...] = bhi - blo + jnp.where(lo, 0, 1024)
        pltpu.async_copy(word_hbm.at[idsv.at[pl.ds(g8, GK)]], rw, sg)
        pltpu.async_copy(x_hbm.at[xi], rx, sg)
        pltpu.async_copy(y_hbm.at[yi], ry, sg)
        pltpu.async_copy(hw_hbm.at[hi], rhw, sg)
        s0 = pl.multiple_of(jnp.bitwise_and(g8, 511), GK)
        pltpu.async_copy(pos_hbm.at[pl.ds(s0, GK)], pb, sg)

    def wait_gathers(rw, rx, ry, rhw, pb, sg):
        pltpu.make_async_copy(word_hbm.at[pl.ds(0, GK)], rw, sg).wait()
        pltpu.make_async_copy(x_hbm.at[pl.ds(0, 2 * GK)], rx, sg).wait()
        pltpu.make_async_copy(y_hbm.at[pl.ds(0, 2 * GK)], ry, sg).wait()
        pltpu.make_async_copy(hw_hbm.at[pl.ds(0, 2 * GK)], rhw, sg).wait()
        pltpu.make_async_copy(pos_hbm.at[pl.ds(0, GK)], pb, sg).wait()

    def wait_out(ac, so):
        pltpu.make_async_copy(ac, out_hbm.at[pl.ds(base, GK)], so).wait()

    def compute(g, rw, rx, ry, rhw, pb, ac):
        g8 = g * GK

        def tbody(t, ct):
            ttf = plsc.load_gather(ttv, [jnp.full((16,), g8 + t, I32)]
                                   ).astype(F32)
            z = jnp.zeros((16,), F32)

            @plsc.parallel_loop(0, NJ, 1, unroll=8, carry=(z, z))
            def sbody(j, carry):
                sv, qv = carry
                sl = pl.ds(j * 16, 16)
                s = pb[t, sl] + ttr[0, sl] + ttf * dlt[sl]
                s = s + rw[t, sl]
                s = s + rx[t, sl] + rx[t + GK, sl]
                s = s + ry[t, sl] + ry[t + GK, sl]
                s = s + rhw[t, sl] + rhw[t + GK, sl]
                ac[t, sl] = s
                return sv + s, qv + s * s

            sv, qv = sbody
            mv = lax.broadcast_in_dim(jnp.sum(sv), (16,), ()) * (1.0 / H)
            q2 = lax.broadcast_in_dim(jnp.sum(qv), (16,), ()) * (1.0 / H)
            var = q2 - mv * mv + 1e-5
            iv = plsc.bitcast(var, I32)
            yv = plsc.bitcast(jnp.int32(0x5F3759DF) - (iv >> 1), F32)
            for _ in range(3):
                yv = yv * (1.5 - 0.5 * var * yv * yv)
            bc = -mv * yv

            @plsc.parallel_loop(0, NJ, 1, unroll=8)
            def nbody(j):
                sl = pl.ds(j * 16, 16)
                ac[t, sl] = (ac[t, sl] * yv + bc) * gv[sl] + bv[sl]
            return ct
        lax.fori_loop(0, GK, tbody, 0)

    def fire_out(g, ac, so):
        off = pl.multiple_of(base + g * GK, GK)
        pltpu.async_copy(ac, out_hbm.at[pl.ds(off, GK)], so)

    fire(0, rw0, rx0, ry0, rhw0, pb0, xi0, yi0, hi0, sg0)

    def ibody(i, c):
        g0 = 2 * i
        g1 = g0 + 1
        fire(g1, rw1, rx1, ry1, rhw1, pb1, xi1, yi1, hi1, sg1)
        wait_gathers(rw0, rx0, ry0, rhw0, pb0, sg0)

        @pl.when(i > 0)
        def _():
            wait_out(ac0, so0)
        compute(g0, rw0, rx0, ry0, rhw0, pb0, ac0)
        fire_out(g0, ac0, so0)

        @pl.when(i < NG // 2 - 1)
        def _():
            fire(g0 + 2, rw0, rx0, ry0, rhw0, pb0, xi0, yi0, hi0, sg0)
        wait_gathers(rw1, rx1, ry1, rhw1, pb1, sg1)

        @pl.when(i > 0)
        def _():
            wait_out(ac1, so1)
        compute(g1, rw1, rx1, ry1, rhw1, pb1, ac1)
        fire_out(g1, ac1, so1)
        return c
    lax.fori_loop(0, NG // 2, ibody, 0)
    wait_out(ac0, so0)
    wait_out(ac1, so1)


def kernel(input_ids, bbox, token_type_ids, word_emb, x_pos_emb, y_pos_emb,
           h_pos_emb, w_pos_emb, token_type_emb, pos_emb, ln_gamma, ln_beta):
    b, s = input_ids.shape
    ids = input_ids.reshape(-1).astype(I32)
    bb = bbox.reshape(-1).astype(I32)
    tt = token_type_ids.reshape(-1).astype(I32)
    hw = jnp.concatenate([h_pos_emb, w_pos_emb], axis=0)
    out = _emb_ln(word_emb, x_pos_emb, y_pos_emb, hw, token_type_emb,
                  pos_emb[:512], ids, bb, tt, ln_gamma, ln_beta)
    return out.reshape(b, s, H)
